# baseline (speedup 1.0000x reference)
import functools

import jax
import jax.numpy as jnp
from jax import lax
from jax.experimental import pallas as pl
from jax.experimental.pallas import tpu as pltpu
from jax.experimental.pallas import tpu_sc as plsc

_N = 10000
_E = 320000
_H = 64
_NC = 2
_NS = 16
_NW = _NC * _NS
_EPW = _E // _NW
_CHG = 80
_NCHG = _EPW // _CHG
_CH = 40
_NCH = _EPW // _CH
_SPAN = 632


def _sc_mesh():
    return plsc.VectorSubcoreMesh(
        core_axis_name="c", subcore_axis_name="s",
        num_cores=_NC, num_subcores=_NS)


def _row_window(s):
    start = (s * (_N // _NS)) // 8 * 8
    return pl.multiple_of(start, 8)


def _zero_rows(ref, nrows, ncols):
    @pl.loop(0, nrows)
    def _(i):
        for j in range(ncols // 16):
            ref[i, pl.ds(j * 16, 16)] = jnp.zeros((16,), jnp.float32)


_DB = 3200


def _hist_body(dst_ref, out_ref):
    dd = jnp.reshape(dst_ref[...], (_DB, 1))
    ohq = (dd // 128 == lax.broadcasted_iota(jnp.int32, (_DB, 80), 1))
    ohr = (dd % 128 == lax.broadcasted_iota(jnp.int32, (_DB, 128), 1))
    part = lax.dot_general(ohq.astype(jnp.float32), ohr.astype(jnp.float32),
                           (((0,), (0,)), ((), ())),
                           preferred_element_type=jnp.float32)

    @pl.when(pl.program_id(0) == 0)
    def _():
        out_ref[...] = jnp.zeros_like(out_ref)

    out_ref[...] += part


def _deg_hist(dst3):
    hist = pl.pallas_call(
        _hist_body,
        grid=(_E // _DB,),
        in_specs=[pl.BlockSpec((1, _DB, 1), lambda i: (i, 0, 0))],
        out_specs=pl.BlockSpec((80, 128), lambda i: (0, 0)),
        out_shape=jax.ShapeDtypeStruct((80, 128), jnp.float32),
    )(dst3)
    return hist.reshape(-1)[:_N]


@functools.partial(
    pl.kernel,
    out_type=jax.ShapeDtypeStruct((_NC, _N, _H), jnp.float32),
    mesh=_sc_mesh(),
    compiler_params=pltpu.CompilerParams(use_tc_tiling_on_sc=False, needs_layout_passes=False),
    scratch_types=[
        pltpu.VMEM((_NCHG, 1, _CHG), jnp.int32),
        pltpu.VMEM((_NCHG, 1, _CHG), jnp.int32),
        pltpu.VMEM((_CHG, _H), jnp.float32),
        pltpu.VMEM((_SPAN, _H), jnp.float32),
        pltpu.VMEM_SHARED((_N, _H), jnp.float32),
        pltpu.SemaphoreType.DMA,
    ],
)
def _gs_kernel(tab_hbm, src_hbm, dst_hbm, out_hbm,
               sidx, didx, rows, zbuf, acc, sem):
    c = lax.axis_index("c")
    s = lax.axis_index("s")
    w = c * _NS + s
    win = _row_window(s)

    _zero_rows(zbuf, _SPAN, _H)
    pltpu.sync_copy(zbuf, acc.at[pl.ds(win, _SPAN)])
    plsc.subcore_barrier()

    pltpu.sync_copy(src_hbm.at[w], sidx)
    pltpu.sync_copy(dst_hbm.at[w], didx)

    @pl.loop(0, _NCHG)
    def _(j):
        pltpu.async_copy(tab_hbm.at[sidx.at[j, 0]], rows, sem).wait()
        pltpu.sync_copy(rows, acc.at[didx.at[j, 0]], add=True)

    plsc.subcore_barrier()
    pltpu.sync_copy(acc.at[pl.ds(win, _SPAN)],
                    out_hbm.at[c, pl.ds(win, _SPAN)])


@functools.partial(
    pl.kernel,
    out_type=(jax.ShapeDtypeStruct((_NC, _N, _H), jnp.float32),
              jax.ShapeDtypeStruct((_E * 4,), jnp.float32)),
    mesh=_sc_mesh(),
    compiler_params=pltpu.CompilerParams(use_tc_tiling_on_sc=False, needs_layout_passes=False),
    scratch_types=[
        pltpu.VMEM((_NCH, 1, _CH), jnp.int32),
        pltpu.VMEM((_NCH, 1, _CH), jnp.int32),
        pltpu.VMEM((_CH, 2 * _H), jnp.float32),
        pltpu.VMEM((_CH, 2 * _H), jnp.float32),
        pltpu.VMEM((_CH, _H), jnp.float32),
        pltpu.VMEM((_CH, _H), jnp.float32),
        pltpu.VMEM((_CH, _H), jnp.float32),
        pltpu.VMEM((_CH * 4,), jnp.float32),
        pltpu.VMEM((_SPAN, _H), jnp.float32),
        pltpu.VMEM_SHARED((_N, _H), jnp.float32),
        pltpu.SemaphoreType.DMA,
        pltpu.SemaphoreType.DMA,
        pltpu.SemaphoreType.DMA,
        pltpu.SemaphoreType.DMA,
    ],
)
def _gat_kernel(hges_hbm, ed_hbm, src_hbm, dst_hbm, outx, outw,
                sidx, didx, hfa, hfb, eda, edb, whb, wflat, zbuf, accx,
                sema0, sema1, semb0, semb1):
    c = lax.axis_index("c")
    s = lax.axis_index("s")
    w = c * _NS + s
    win = _row_window(s)

    _zero_rows(zbuf, _SPAN, _H)
    pltpu.sync_copy(zbuf, accx.at[pl.ds(win, _SPAN)])
    plsc.subcore_barrier()

    pltpu.sync_copy(src_hbm.at[w], sidx)
    pltpu.sync_copy(dst_hbm.at[w], didx)

    lane = lax.broadcasted_iota(jnp.int32, (16,), 0)
    grow = lane // 4
    gcol = (2 * _H // 2) + (lane % 4) * 16

    def issue(j, hbuf, ebuf, s0, s1):
        pltpu.async_copy(hges_hbm.at[sidx.at[j, 0]], hbuf, s0)
        pltpu.async_copy(ed_hbm.at[didx.at[j, 0]], ebuf, s1)

    def drain(hbuf, ebuf, s0, s1):
        pltpu.make_async_copy(hges_hbm.at[sidx.at[0, 0]], hbuf, s0).wait()
        pltpu.make_async_copy(ed_hbm.at[didx.at[0, 0]], ebuf, s1).wait()

    def work(j, hbuf, ebuf):
        @pl.loop(0, _CH)
        def _(i):
            for k in range(_H // 16):
                sl = pl.ds(k * 16, 16)
                sl2 = pl.ds(_H + k * 16, 16)
                z = hbuf[i, sl2] + ebuf[i, sl]
                z = jnp.where(z > 0.0, z, 0.2 * z)
                wv = jnp.exp(z)
                hbuf[i, sl2] = wv
                whb[i, sl] = hbuf[i, sl] * wv

        @pl.loop(0, _CH // 4)
        def _(g):
            wflat[pl.ds(g * 16, 16)] = plsc.load_gather(
                hbuf, [g * 4 + grow, gcol])

        pltpu.sync_copy(whb, accx.at[didx.at[j, 0]], add=True)
        ebase = pl.multiple_of((w * _EPW + j * _CH) * 4, 8)
        pltpu.sync_copy(wflat, outw.at[pl.ds(ebase, _CH * 4)])

    issue(0, hfa, eda, sema0, sema1)

    @pl.loop(0, _NCH, step=2)
    def _(j):
        drain(hfa, eda, sema0, sema1)
        issue(j + 1, hfb, edb, semb0, semb1)
        work(j, hfa, eda)
        drain(hfb, edb, semb0, semb1)

        @pl.when(j + 2 < _NCH)
        def _():
            issue(j + 2, hfa, eda, sema0, sema1)

        work(j + 1, hfb, edb)

    plsc.subcore_barrier()
    pltpu.sync_copy(accx.at[pl.ds(win, _SPAN)],
                    outx.at[c, pl.ds(win, _SPAN)])


def _den_body(dst_ref, w_ref, out_ref):
    dd = jnp.reshape(dst_ref[...], (_DB, 1))
    wv = jnp.reshape(w_ref[...], (_DB, 4))
    ohq = (dd // 128 == lax.broadcasted_iota(jnp.int32, (_DB, 80), 1))
    ohr = (dd % 128 == lax.broadcasted_iota(jnp.int32, (_DB, 128), 1))
    ohqf = ohq.astype(jnp.float32)
    ohrf = ohr.astype(jnp.float32)

    @pl.when(pl.program_id(0) == 0)
    def _():
        out_ref[...] = jnp.zeros_like(out_ref)

    for h in range(4):
        part = lax.dot_general(ohqf, ohrf * wv[:, h:h + 1],
                               (((0,), (0,)), ((), ())),
                               preferred_element_type=jnp.float32)
        out_ref[h] += part


def _den_hist(dst3, we):
    hist = pl.pallas_call(
        _den_body,
        grid=(_E // _DB,),
        in_specs=[pl.BlockSpec((1, _DB, 1), lambda i: (i, 0, 0)),
                  pl.BlockSpec((1, _DB, 4), lambda i: (i, 0, 0))],
        out_specs=pl.BlockSpec((4, 80, 128), lambda i: (0, 0, 0)),
        out_shape=jax.ShapeDtypeStruct((4, 80, 128), jnp.float32),
    )(dst3, we.reshape(_E // _DB, _DB, 4))
    return hist.reshape(4, -1)[:, :_N]


def _gin_body(ginp_ref, x2_ref, x1_ref, eps_ref, gw1_ref, gb1_ref,
              gg_ref, gb_ref, gw2_ref, gb2_ref, g3_ref, bb3_ref, x3_ref):
    gv = (1.0 + eps_ref[0, 0]) * x2_ref[...] + ginp_ref[0] + ginp_ref[1]
    gv = _mm(gv, gw1_ref[...]) + gb1_ref[...]
    gv = _bn_relu(gv, gg_ref[...], gb_ref[...])
    gv = _mm(gv, gw2_ref[...]) + gb2_ref[...]
    x3_ref[...] = _bn_relu(gv + x1_ref[...], g3_ref[...], bb3_ref[...])


def _tcd_body(x1_ref, x2_ref, x3_ref, b_ref, pw_ref, pb_ref, out_ref):
    x1 = x1_ref[...]
    x2 = x2_ref[...]
    bcol = b_ref[...]
    gi = lax.broadcasted_iota(jnp.int32, (_N, 64), 1)
    oh = (bcol == gi).astype(jnp.float32)
    cnt = jnp.maximum(jnp.sum(oh, axis=0), 1.0)[:, None]

    neg = jnp.float32(-jnp.inf)
    bblk = jnp.reshape(bcol, (_N // 8, 8, 1))
    blo = jnp.min(bblk, axis=1)
    bhi = jnp.max(bblk, axis=1)

    def seg(xv):
        ssum = lax.dot_general(oh, xv, (((0,), (0,)), ((), ())),
                               preferred_element_type=jnp.float32)
        rows = lax.broadcasted_iota(jnp.int32, (64, 64), 0)
        xb = jnp.reshape(xv, (_N // 8, 8, _H))
        mlo = jnp.max(jnp.where(bblk == blo[:, None], xb, neg), axis=1)
        mhi = jnp.max(jnp.where(bblk == bhi[:, None], xb, neg), axis=1)

        def body(g, acc):
            c1 = jnp.max(jnp.where(blo == g, mlo, neg), axis=0)
            c2 = jnp.max(jnp.where(bhi == g, mhi, neg), axis=0)
            mvals = jnp.maximum(c1, c2)
            return jnp.where(rows == g, mvals[None, :], acc)

        smax = lax.fori_loop(0, 64, body, jnp.full((64, 64), neg))
        return (ssum / cnt + smax + ssum) / 3.0

    pw = pw_ref[...]
    out_ref[...] = (_mm(seg(x1), pw[0:64]) + _mm(seg(x2), pw[64:128])
                    + _mm(seg(x3_ref[...]), pw[128:192]) + pb_ref[...])


def _tcd(ginp, x2, x1, gin_eps, gin_W1, gin_b1, gin_bn_g, gin_bn_b,
         gin_W2, gin_b2, bn3_g, bn3_b, batch, proj_W, proj_b):
    x3 = pl.pallas_call(
        _gin_body,
        out_shape=jax.ShapeDtypeStruct((_N, _H), jnp.float32),
    )(ginp, x2, x1, gin_eps.reshape(1, 1), gin_W1,
      gin_b1.reshape(1, 2 * _H), gin_bn_g.reshape(1, 2 * _H),
      gin_bn_b.reshape(1, 2 * _H), gin_W2, gin_b2.reshape(1, _H),
      bn3_g.reshape(1, _H), bn3_b.reshape(1, _H))
    return pl.pallas_call(
        _tcd_body,
        out_shape=jax.ShapeDtypeStruct((64, _H), jnp.float32),
        compiler_params=pltpu.CompilerParams(
            vmem_limit_bytes=64 * 1024 * 1024),
    )(x1, x2, x3, batch.reshape(_N, 1), proj_W, proj_b.reshape(1, _H))


def _bn_relu(v, g, b):
    m = jnp.mean(v, axis=0)
    c = v - m
    var = jnp.mean(c * c, axis=0)
    return jnp.maximum(c / jnp.sqrt(var + 1e-5) * g + b, 0.0)


_DN = (((1,), (0,)), ((), ()))


def _mm(a, b):
    return lax.dot_general(a, b, _DN, preferred_element_type=jnp.float32)


def _tca_body(x_ref, w1_ref, deg_ref, h_ref, hsc_ref):
    dinv = lax.rsqrt(deg_ref[...])
    h = _mm(x_ref[...], w1_ref[...])
    h_ref[...] = h
    hsc_ref[...] = h * dinv


def _tca(x, W1, deg2):
    return pl.pallas_call(
        _tca_body,
        out_shape=(jax.ShapeDtypeStruct((_N, _H), jnp.float32),
                   jax.ShapeDtypeStruct((_N, _H), jnp.float32)),
    )(x, W1, deg2)


def _tcb_body(gp_ref, h_ref, deg_ref, b1_ref, g1_ref, bb1_ref, w2_ref,
              asr_ref, adr_ref, x1_ref, hg_ref, hges_ref, edr_ref, ss_ref):
    dinv = lax.rsqrt(deg_ref[...])
    x1 = (dinv * (gp_ref[0] + gp_ref[1]) + dinv * dinv * h_ref[...]
          + b1_ref[...])
    x1 = _bn_relu(x1, g1_ref[...], bb1_ref[...])
    x1_ref[...] = x1
    hg = _mm(x1, w2_ref[...])
    hg_ref[...] = hg
    rows = lax.broadcasted_iota(jnp.int32, (_H, _H), 0) // 16
    cols = lax.broadcasted_iota(jnp.int32, (_H, _H), 1) // 16
    bs = (rows == cols).astype(jnp.float32)
    esr = _mm(hg * asr_ref[...], bs)
    edr = _mm(hg * adr_ref[...], bs)
    hges_ref[...] = jnp.concatenate([hg, esr], axis=1)
    edr_ref[...] = edr
    zs = esr + edr
    ss_ref[...] = jnp.exp(jnp.where(zs > 0.0, zs, 0.2 * zs))


def _tcb(gp, h, deg2, b1, bn1_g, bn1_b, W2, att_src, att_dst):
    sds = jax.ShapeDtypeStruct((_N, _H), jnp.float32)
    sds2 = jax.ShapeDtypeStruct((_N, 2 * _H), jnp.float32)
    return pl.pallas_call(
        _tcb_body,
        out_shape=(sds, sds, sds2, sds, sds),
    )(gp, h, deg2, b1.reshape(1, _H), bn1_g.reshape(1, _H),
      bn1_b.reshape(1, _H), W2, att_src.reshape(1, _H),
      att_dst.reshape(1, _H))


def _tcc_body(accx_ref, hg_ref, ss_ref, dent_ref, b2_ref, g2_ref, bb2_ref,
              x2_ref):
    hg = hg_ref[...]
    ss = ss_ref[...]
    num = accx_ref[0] + accx_ref[1] + hg * ss
    dent = dent_ref[...]
    den = jnp.concatenate(
        [jnp.broadcast_to(dent[:, h:h + 1], (_N, 16)) for h in range(4)],
        axis=1) + ss
    x2 = num / (den + 1e-16) + b2_ref[...]
    x2_ref[...] = _bn_relu(x2, g2_ref[...], bb2_ref[...])


def _tcc(accx, hg, sself, dent, b2, bn2_g, bn2_b):
    return pl.pallas_call(
        _tcc_body,
        out_shape=jax.ShapeDtypeStruct((_N, _H), jnp.float32),
    )(accx, hg, sself, dent, b2.reshape(1, _H), bn2_g.reshape(1, _H),
      bn2_b.reshape(1, _H))


def kernel(x, edge_index, batch, W1, b1, bn1_g, bn1_b, W2, att_src,
           att_dst, b2, bn2_g, bn2_b, gin_eps, gin_W1, gin_b1, gin_bn_g,
           gin_bn_b, gin_W2, gin_b2, bn3_g, bn3_b, proj_W, proj_b):
    src4g = edge_index[0].reshape(_NW, _NCHG, 1, _CHG)
    dst4g = edge_index[1].reshape(_NW, _NCHG, 1, _CHG)
    src4 = edge_index[0].reshape(_NW, _NCH, 1, _CH)
    dst4 = edge_index[1].reshape(_NW, _NCH, 1, _CH)
    dst3 = edge_index[1].reshape(_E // _DB, _DB, 1)

    deg2 = (1.0 + _deg_hist(dst3)).reshape(_N, 1)
    h, hsc = _tca(x, W1, deg2)
    gp = _gs_kernel(hsc, src4g, dst4g)
    x1, hg, hges, edr, sself = _tcb(gp, h, deg2, b1, bn1_g, bn1_b, W2,
                                    att_src, att_dst)

    accx, we = _gat_kernel(hges, edr, src4, dst4)
    dent = _den_hist(dst3, we).T
    x2 = _tcc(accx, hg, sself, dent, b2, bn2_g, bn2_b)

    ginp = _gs_kernel(x2, src4g, dst4g)
    return _tcd(ginp, x2, x1, gin_eps, gin_W1, gin_b1, gin_bn_g,
                gin_bn_b, gin_W2, gin_b2, bn3_g, bn3_b, batch,
                proj_W, proj_b)


# speedup vs baseline: 1.1135x; 1.1135x over previous
import functools

import jax
import jax.numpy as jnp
from jax import lax
from jax.experimental import pallas as pl
from jax.experimental.pallas import tpu as pltpu
from jax.experimental.pallas import tpu_sc as plsc

_N = 10000
_E = 320000
_H = 64
_NC = 2
_NS = 16
_NW = _NC * _NS
_EPW = _E // _NW
_CHG = 80
_NCHG = _EPW // _CHG
_CH = 40
_NCH = _EPW // _CH
_SPAN = 632


def _sc_mesh():
    return plsc.VectorSubcoreMesh(
        core_axis_name="c", subcore_axis_name="s",
        num_cores=_NC, num_subcores=_NS)


def _row_window(s):
    start = (s * (_N // _NS)) // 8 * 8
    return pl.multiple_of(start, 8)


def _zero_rows(ref, nrows, ncols):
    @pl.loop(0, nrows)
    def _(i):
        for j in range(ncols // 16):
            ref[i, pl.ds(j * 16, 16)] = jnp.zeros((16,), jnp.float32)


_DB = 3200


def _hist_body(dst_ref, out_ref):
    dd = jnp.reshape(dst_ref[...], (_DB, 1))
    ohq = (dd // 128 == lax.broadcasted_iota(jnp.int32, (_DB, 80), 1))
    ohr = (dd % 128 == lax.broadcasted_iota(jnp.int32, (_DB, 128), 1))
    part = lax.dot_general(ohq.astype(jnp.float32), ohr.astype(jnp.float32),
                           (((0,), (0,)), ((), ())),
                           preferred_element_type=jnp.float32)

    @pl.when(pl.program_id(0) == 0)
    def _():
        out_ref[...] = jnp.zeros_like(out_ref)

    out_ref[...] += part


def _deg_hist(dst3):
    hist = pl.pallas_call(
        _hist_body,
        grid=(_E // _DB,),
        in_specs=[pl.BlockSpec((1, _DB, 1), lambda i: (i, 0, 0))],
        out_specs=pl.BlockSpec((80, 128), lambda i: (0, 0)),
        out_shape=jax.ShapeDtypeStruct((80, 128), jnp.float32),
    )(dst3)
    return hist.reshape(-1)[:_N]


@functools.partial(
    pl.kernel,
    out_type=jax.ShapeDtypeStruct((_NC, _N, _H), jnp.float32),
    mesh=_sc_mesh(),
    compiler_params=pltpu.CompilerParams(use_tc_tiling_on_sc=False, needs_layout_passes=False),
    scratch_types=[
        pltpu.VMEM((_NCHG, 1, _CHG), jnp.int32),
        pltpu.VMEM((_NCHG, 1, _CHG), jnp.int32),
        pltpu.VMEM((_CHG, _H), jnp.float32),
        pltpu.VMEM((_SPAN, _H), jnp.float32),
        pltpu.VMEM_SHARED((_N, _H), jnp.float32),
        pltpu.SemaphoreType.DMA,
    ],
)
def _gs_kernel(tab_hbm, src_hbm, dst_hbm, out_hbm,
               sidx, didx, rows, zbuf, acc, sem):
    c = lax.axis_index("c")
    s = lax.axis_index("s")
    w = c * _NS + s
    win = _row_window(s)

    _zero_rows(zbuf, _SPAN, _H)
    pltpu.sync_copy(zbuf, acc.at[pl.ds(win, _SPAN)])
    plsc.subcore_barrier()

    pltpu.sync_copy(src_hbm.at[w], sidx)
    pltpu.sync_copy(dst_hbm.at[w], didx)

    @pl.loop(0, _NCHG)
    def _(j):
        pltpu.async_copy(tab_hbm.at[sidx.at[j, 0]], rows, sem).wait()
        pltpu.sync_copy(rows, acc.at[didx.at[j, 0]], add=True)

    plsc.subcore_barrier()
    pltpu.sync_copy(acc.at[pl.ds(win, _SPAN)],
                    out_hbm.at[c, pl.ds(win, _SPAN)])


@functools.partial(
    pl.kernel,
    out_type=(jax.ShapeDtypeStruct((_NC, _N, _H), jnp.float32),
              jax.ShapeDtypeStruct((_E * 4,), jnp.float32)),
    mesh=_sc_mesh(),
    compiler_params=pltpu.CompilerParams(use_tc_tiling_on_sc=False, needs_layout_passes=False),
    scratch_types=[
        pltpu.VMEM((_NCH, 1, _CH), jnp.int32),
        pltpu.VMEM((_NCH, 1, _CH), jnp.int32),
        pltpu.VMEM((_CH, 2 * _H), jnp.float32),
        pltpu.VMEM((_CH, 2 * _H), jnp.float32),
        pltpu.VMEM((_CH, _H), jnp.float32),
        pltpu.VMEM((_CH, _H), jnp.float32),
        pltpu.VMEM((_CH, _H), jnp.float32),
        pltpu.VMEM((_CH * 4,), jnp.float32),
        pltpu.VMEM((_SPAN, _H), jnp.float32),
        pltpu.VMEM_SHARED((_N, _H), jnp.float32),
        pltpu.SemaphoreType.DMA,
        pltpu.SemaphoreType.DMA,
        pltpu.SemaphoreType.DMA,
        pltpu.SemaphoreType.DMA,
    ],
)
def _gat_kernel(hges_hbm, ed_hbm, src_hbm, dst_hbm, outx, outw,
                sidx, didx, hfa, hfb, eda, edb, whb, wflat, zbuf, accx,
                sema0, sema1, semb0, semb1):
    c = lax.axis_index("c")
    s = lax.axis_index("s")
    w = c * _NS + s
    win = _row_window(s)

    _zero_rows(zbuf, _SPAN, _H)
    pltpu.sync_copy(zbuf, accx.at[pl.ds(win, _SPAN)])
    plsc.subcore_barrier()

    pltpu.sync_copy(src_hbm.at[w], sidx)
    pltpu.sync_copy(dst_hbm.at[w], didx)

    lane = lax.broadcasted_iota(jnp.int32, (16,), 0)
    grow = lane // 4
    gcol = (2 * _H // 2) + (lane % 4) * 16

    def issue(j, hbuf, ebuf, s0, s1):
        pltpu.async_copy(hges_hbm.at[sidx.at[j, 0]], hbuf, s0)
        pltpu.async_copy(ed_hbm.at[didx.at[j, 0]], ebuf, s1)

    def drain(hbuf, ebuf, s0, s1):
        pltpu.make_async_copy(hges_hbm.at[sidx.at[0, 0]], hbuf, s0).wait()
        pltpu.make_async_copy(ed_hbm.at[didx.at[0, 0]], ebuf, s1).wait()

    def work(j, hbuf, ebuf):
        @pl.loop(0, _CH)
        def _(i):
            for k in range(_H // 16):
                sl = pl.ds(k * 16, 16)
                sl2 = pl.ds(_H + k * 16, 16)
                z = hbuf[i, sl2] + ebuf[i, sl]
                z = jnp.where(z > 0.0, z, 0.2 * z)
                wv = jnp.exp(z)
                hbuf[i, sl2] = wv
                whb[i, sl] = hbuf[i, sl] * wv

        @pl.loop(0, _CH // 4)
        def _(g):
            wflat[pl.ds(g * 16, 16)] = plsc.load_gather(
                hbuf, [g * 4 + grow, gcol])

        pltpu.sync_copy(whb, accx.at[didx.at[j, 0]], add=True)
        ebase = pl.multiple_of((w * _EPW + j * _CH) * 4, 8)
        pltpu.sync_copy(wflat, outw.at[pl.ds(ebase, _CH * 4)])

    issue(0, hfa, eda, sema0, sema1)

    @pl.loop(0, _NCH, step=2)
    def _(j):
        drain(hfa, eda, sema0, sema1)
        issue(j + 1, hfb, edb, semb0, semb1)
        work(j, hfa, eda)
        drain(hfb, edb, semb0, semb1)

        @pl.when(j + 2 < _NCH)
        def _():
            issue(j + 2, hfa, eda, sema0, sema1)

        work(j + 1, hfb, edb)

    plsc.subcore_barrier()
    pltpu.sync_copy(accx.at[pl.ds(win, _SPAN)],
                    outx.at[c, pl.ds(win, _SPAN)])


def _den_body(dst_ref, w_ref, out_ref):
    dd = jnp.reshape(dst_ref[...], (_DB, 1))
    wv = jnp.reshape(w_ref[...], (_DB, 4))
    ohq = (dd // 128 == lax.broadcasted_iota(jnp.int32, (_DB, 80), 1))
    ohr = (dd % 128 == lax.broadcasted_iota(jnp.int32, (_DB, 128), 1))
    ohqf = ohq.astype(jnp.float32)
    ohrf = ohr.astype(jnp.float32)

    @pl.when(pl.program_id(0) == 0)
    def _():
        out_ref[...] = jnp.zeros_like(out_ref)

    for h in range(4):
        part = lax.dot_general(ohqf, ohrf * wv[:, h:h + 1],
                               (((0,), (0,)), ((), ())),
                               preferred_element_type=jnp.float32)
        out_ref[h] += part


def _den_hist(dst3, we):
    hist = pl.pallas_call(
        _den_body,
        grid=(_E // _DB,),
        in_specs=[pl.BlockSpec((1, _DB, 1), lambda i: (i, 0, 0)),
                  pl.BlockSpec((1, _DB, 4), lambda i: (i, 0, 0))],
        out_specs=pl.BlockSpec((4, 80, 128), lambda i: (0, 0, 0)),
        out_shape=jax.ShapeDtypeStruct((4, 80, 128), jnp.float32),
    )(dst3, we.reshape(_E // _DB, _DB, 4))
    return hist.reshape(4, -1)[:, :_N]


def _tcd_body(ginp_ref, x2_ref, x1_ref, eps_ref, gw1_ref, gb1_ref,
              gg_ref, gb_ref, gw2_ref, gb2_ref, g3_ref, bb3_ref,
              b_ref, pw_ref, pb_ref, out_ref):
    x1 = x1_ref[...]
    x2 = x2_ref[...]
    gv = (1.0 + eps_ref[0, 0]) * x2 + ginp_ref[0] + ginp_ref[1]
    gv = _mm(gv, gw1_ref[...]) + gb1_ref[...]
    gv = _bn_relu(gv, gg_ref[...], gb_ref[...])
    gv = _mm(gv, gw2_ref[...]) + gb2_ref[...]
    x3 = _bn_relu(gv + x1, g3_ref[...], bb3_ref[...])

    bcol = b_ref[...]
    gi = lax.broadcasted_iota(jnp.int32, (_N, 64), 1)
    oh = (bcol == gi).astype(jnp.float32)
    cnt = jnp.maximum(jnp.sum(oh, axis=0), 1.0)[:, None]

    neg = jnp.float32(-jnp.inf)

    def seg(xv):
        ssum = lax.dot_general(oh, xv, (((0,), (0,)), ((), ())),
                               preferred_element_type=jnp.float32)
        rows = lax.broadcasted_iota(jnp.int32, (64, 64), 0)

        def body(g, acc):
            mvals = jnp.max(jnp.where(bcol == g, xv, neg), axis=0)
            return jnp.where(rows == g, mvals[None, :], acc)

        smax = lax.fori_loop(0, 64, body, jnp.full((64, 64), neg))
        return (ssum / cnt + smax + ssum) / 3.0

    pw = pw_ref[...]
    out_ref[...] = (_mm(seg(x1), pw[0:64]) + _mm(seg(x2), pw[64:128])
                    + _mm(seg(x3), pw[128:192]) + pb_ref[...])


def _tcd(ginp, x2, x1, gin_eps, gin_W1, gin_b1, gin_bn_g, gin_bn_b,
         gin_W2, gin_b2, bn3_g, bn3_b, batch, proj_W, proj_b):
    return pl.pallas_call(
        _tcd_body,
        out_shape=jax.ShapeDtypeStruct((64, _H), jnp.float32),
    )(ginp, x2, x1, gin_eps.reshape(1, 1), gin_W1,
      gin_b1.reshape(1, 2 * _H), gin_bn_g.reshape(1, 2 * _H),
      gin_bn_b.reshape(1, 2 * _H), gin_W2, gin_b2.reshape(1, _H),
      bn3_g.reshape(1, _H), bn3_b.reshape(1, _H), batch.reshape(_N, 1),
      proj_W, proj_b.reshape(1, _H))


def _bn_relu(v, g, b):
    m = jnp.mean(v, axis=0)
    c = v - m
    var = jnp.mean(c * c, axis=0)
    return jnp.maximum(c / jnp.sqrt(var + 1e-5) * g + b, 0.0)


_DN = (((1,), (0,)), ((), ()))


def _mm(a, b):
    return lax.dot_general(a, b, _DN, preferred_element_type=jnp.float32)


def _tca_body(x_ref, w1_ref, deg_ref, h_ref, hsc_ref):
    dinv = lax.rsqrt(deg_ref[...])
    h = _mm(x_ref[...], w1_ref[...])
    h_ref[...] = h
    hsc_ref[...] = h * dinv


def _tca(x, W1, deg2):
    return pl.pallas_call(
        _tca_body,
        out_shape=(jax.ShapeDtypeStruct((_N, _H), jnp.float32),
                   jax.ShapeDtypeStruct((_N, _H), jnp.float32)),
    )(x, W1, deg2)


def _tcb_body(gp_ref, h_ref, deg_ref, b1_ref, g1_ref, bb1_ref, w2_ref,
              asr_ref, adr_ref, x1_ref, hg_ref, hges_ref, edr_ref, ss_ref):
    dinv = lax.rsqrt(deg_ref[...])
    x1 = (dinv * (gp_ref[0] + gp_ref[1]) + dinv * dinv * h_ref[...]
          + b1_ref[...])
    x1 = _bn_relu(x1, g1_ref[...], bb1_ref[...])
    x1_ref[...] = x1
    hg = _mm(x1, w2_ref[...])
    hg_ref[...] = hg
    rows = lax.broadcasted_iota(jnp.int32, (_H, _H), 0) // 16
    cols = lax.broadcasted_iota(jnp.int32, (_H, _H), 1) // 16
    bs = (rows == cols).astype(jnp.float32)
    esr = _mm(hg * asr_ref[...], bs)
    edr = _mm(hg * adr_ref[...], bs)
    hges_ref[...] = jnp.concatenate([hg, esr], axis=1)
    edr_ref[...] = edr
    zs = esr + edr
    ss_ref[...] = jnp.exp(jnp.where(zs > 0.0, zs, 0.2 * zs))


def _tcb(gp, h, deg2, b1, bn1_g, bn1_b, W2, att_src, att_dst):
    sds = jax.ShapeDtypeStruct((_N, _H), jnp.float32)
    sds2 = jax.ShapeDtypeStruct((_N, 2 * _H), jnp.float32)
    return pl.pallas_call(
        _tcb_body,
        out_shape=(sds, sds, sds2, sds, sds),
    )(gp, h, deg2, b1.reshape(1, _H), bn1_g.reshape(1, _H),
      bn1_b.reshape(1, _H), W2, att_src.reshape(1, _H),
      att_dst.reshape(1, _H))


def _tcc_body(accx_ref, hg_ref, ss_ref, dent_ref, b2_ref, g2_ref, bb2_ref,
              x2_ref):
    hg = hg_ref[...]
    ss = ss_ref[...]
    num = accx_ref[0] + accx_ref[1] + hg * ss
    dent = dent_ref[...]
    den = jnp.concatenate(
        [jnp.broadcast_to(dent[:, h:h + 1], (_N, 16)) for h in range(4)],
        axis=1) + ss
    x2 = num / (den + 1e-16) + b2_ref[...]
    x2_ref[...] = _bn_relu(x2, g2_ref[...], bb2_ref[...])


def _tcc(accx, hg, sself, dent, b2, bn2_g, bn2_b):
    return pl.pallas_call(
        _tcc_body,
        out_shape=jax.ShapeDtypeStruct((_N, _H), jnp.float32),
    )(accx, hg, sself, dent, b2.reshape(1, _H), bn2_g.reshape(1, _H),
      bn2_b.reshape(1, _H))


def kernel(x, edge_index, batch, W1, b1, bn1_g, bn1_b, W2, att_src,
           att_dst, b2, bn2_g, bn2_b, gin_eps, gin_W1, gin_b1, gin_bn_g,
           gin_bn_b, gin_W2, gin_b2, bn3_g, bn3_b, proj_W, proj_b):
    src4g = edge_index[0].reshape(_NW, _NCHG, 1, _CHG)
    dst4g = edge_index[1].reshape(_NW, _NCHG, 1, _CHG)
    src4 = edge_index[0].reshape(_NW, _NCH, 1, _CH)
    dst4 = edge_index[1].reshape(_NW, _NCH, 1, _CH)
    dst3 = edge_index[1].reshape(_E // _DB, _DB, 1)

    deg2 = (1.0 + _deg_hist(dst3)).reshape(_N, 1)
    h, hsc = _tca(x, W1, deg2)
    gp = _gs_kernel(hsc, src4g, dst4g)
    x1, hg, hges, edr, sself = _tcb(gp, h, deg2, b1, bn1_g, bn1_b, W2,
                                    att_src, att_dst)

    accx, we = _gat_kernel(hges, edr, src4, dst4)
    dent = _den_hist(dst3, we).T
    x2 = _tcc(accx, hg, sself, dent, b2, bn2_g, bn2_b)

    ginp = _gs_kernel(x2, src4g, dst4g)
    return _tcd(ginp, x2, x1, gin_eps, gin_W1, gin_b1, gin_bn_g,
                gin_bn_b, gin_W2, gin_b2, bn3_g, bn3_b, batch,
                proj_W, proj_b)
